# baseline (speedup 1.0000x reference)
import jax
import jax.numpy as jnp
from jax import lax
from jax.experimental import pallas as pl
from jax.experimental.pallas import tpu as pltpu
from jax.experimental.pallas import tpu_sc as plsc

_B = 16384
_L = 50
_DT = 32
_DC = 16
_VT = 1000000
_VC = 100000
_NC = 2
_NS = 16
_NW = _NC * _NS
_LP = 56
_CH = 2
_CL = _CH * _LP
_RPT = _B // _NW
_IPT = _RPT * _LP
_CPT = _RPT // _CH
_CPW = 2 * _CPT
_SPW = 2 * _RPT
_NBUF = 4

_CAT_CH = 128
_CAT_PW = _B // _NW
_CAT_CPW = _CAT_PW // _CAT_CH


def _tree_sum(parts):
    while len(parts) > 1:
        nxt = [parts[i] + parts[i + 1] for i in range(0, len(parts) - 1, 2)]
        if len(parts) % 2:
            nxt.append(parts[-1])
        parts = nxt
    return parts[0]


def _detile_body(in_ref, out_ref):
    x = in_ref[...]
    xt = x.T
    ns = 128 // x.shape[0]
    sr = x.shape[1] // ns
    out_ref[...] = jnp.concatenate(
        [xt[a * sr:(a + 1) * sr] for a in range(ns)], axis=1)


def _detile(table_t):
    d, v = table_t.shape
    blkv = 4096
    nblk = pl.cdiv(v, blkv)
    brows = blkv * d // 128
    return pl.pallas_call(
        _detile_body,
        grid=(nblk,),
        in_specs=[pl.BlockSpec((d, blkv), lambda i: (0, i))],
        out_specs=pl.BlockSpec((brows, 128), lambda i: (i, 0)),
        out_shape=jax.ShapeDtypeStruct((nblk * brows, 128), jnp.float32),
    )(table_t)


def _sc_body(qidx_hbm, pidx_hbm, cat_hbm, table_hbm, cat_table_hbm,
             means_hbm, cat_out_hbm,
             idx_v, rows_v, means_v, cidx_v, crows_v, gsems, csem):
    wid = lax.axis_index("s") * _NC + lax.axis_index("c")

    pltpu.sync_copy(qidx_hbm.at[pl.ds(wid * _IPT, _IPT)],
                    idx_v.at[pl.ds(0, _IPT)])
    pltpu.sync_copy(pidx_hbm.at[pl.ds(wid * _IPT, _IPT)],
                    idx_v.at[pl.ds(_IPT, _IPT)])
    pltpu.sync_copy(cat_hbm.at[pl.ds(wid * _CAT_PW, _CAT_PW)], cidx_v)

    for j in range(_CAT_CPW):
        pltpu.async_copy(
            cat_table_hbm.at[cidx_v.at[pl.ds(j * _CAT_CH, _CAT_CH)]],
            crows_v.at[pl.ds(j * _CAT_CH, _CAT_CH)], csem)

    lane = lax.iota(jnp.int32, 16)
    tail_mask = lane >= 14

    def start(c, b):
        pltpu.async_copy(table_hbm.at[idx_v.at[pl.ds(c * _CL, _CL)]],
                         rows_v.at[b], gsems[b])

    def wait(c, b):
        pltpu.make_async_copy(table_hbm.at[idx_v.at[pl.ds(c * _CL, _CL)]],
                              rows_v.at[b], gsems[b]).wait()

    def accum(c, b):
        rb = rows_v.at[b]
        for s in range(_CH):
            base = s * _LP
            lo = _tree_sum([rb[base + j, pl.ds(0, 16)] for j in range(_L)])
            hi = _tree_sum([rb[base + j, pl.ds(16, 16)] for j in range(_L)])
            off = c * _CL + base
            pc = plsc.all_reduce_population_count(
                idx_v[pl.ds(off, 16)] != 0)
            pc = pc + plsc.all_reduce_population_count(
                idx_v[pl.ds(off + 16, 16)] != 0)
            pc = pc + plsc.all_reduce_population_count(
                idx_v[pl.ds(off + 32, 16)] != 0)
            pc = pc + plsc.all_reduce_population_count(
                (idx_v[pl.ds(off + 34, 16)] != 0) & tail_mask)
            inv = 1.0 / jnp.maximum(pc, 1).astype(jnp.float32)
            row = c * _CH + s
            means_v[row, pl.ds(0, 16)] = lo * inv
            means_v[row, pl.ds(16, 16)] = hi * inv

    for b in range(_NBUF - 1):
        start(b, b)

    @pl.loop(0, _CPW, step=_NBUF)
    def _(c0):
        for b in range(_NBUF):
            c = c0 + b
            nxt = c + (_NBUF - 1)

            @pl.when(nxt < _CPW)
            def _():
                start(nxt, (b + _NBUF - 1) % _NBUF)

            wait(c, b)
            accum(c, b)

    half = _SPW // 2
    pltpu.sync_copy(means_v.at[pl.ds(0, half)],
                    means_hbm.at[pl.ds(wid * half, half)])
    pltpu.sync_copy(means_v.at[pl.ds(half, half)],
                    means_hbm.at[pl.ds(_B + wid * half, half)])

    for j in range(_CAT_CPW):
        pltpu.make_async_copy(
            cat_table_hbm.at[cidx_v.at[pl.ds(j * _CAT_CH, _CAT_CH)]],
            crows_v.at[pl.ds(j * _CAT_CH, _CAT_CH)], csem).wait()
    pltpu.sync_copy(crows_v, cat_out_hbm.at[pl.ds(wid * _CAT_PW, _CAT_PW)])


def _sc_gather():
    return pl.kernel(
        _sc_body,
        out_type=(jax.ShapeDtypeStruct((2 * _B, _DT), jnp.float32),
                  jax.ShapeDtypeStruct((_B, _DC), jnp.float32)),
        mesh=plsc.VectorSubcoreMesh(core_axis_name="c", subcore_axis_name="s"),
        compiler_params=pltpu.CompilerParams(use_tc_tiling_on_sc=False,
                                             needs_layout_passes=False),
        scratch_types=[
            pltpu.VMEM((2 * _IPT,), jnp.int32),
            pltpu.VMEM((_NBUF, _CL, _DT), jnp.float32),
            pltpu.VMEM((_SPW, _DT), jnp.float32),
            pltpu.VMEM((_CAT_PW,), jnp.int32),
            pltpu.VMEM((_CAT_PW, _DC), jnp.float32),
            [pltpu.SemaphoreType.DMA] * _NBUF,
            pltpu.SemaphoreType.DMA,
        ],
    )


_BLK = 2048


def _tc_body(qm_ref, pm_ref, cat_ref,
             qw0, qb0, qw1, qb1, qw2, qb2,
             pw0t, pw0c, pb0, pw1, pb1, pw2, pb2,
             qo_ref, po_ref):
    q = qm_ref[...]
    h = jnp.maximum(
        jnp.dot(q, qw0[...], preferred_element_type=jnp.float32) + qb0[...],
        0.0)
    h = jnp.maximum(
        jnp.dot(h, qw1[...], preferred_element_type=jnp.float32) + qb1[...],
        0.0)
    qo_ref[...] = (jnp.dot(h, qw2[...], preferred_element_type=jnp.float32)
                   + qb2[...])

    t = pm_ref[...]
    h = (jnp.dot(t, pw0t[...], preferred_element_type=jnp.float32)
         + jnp.dot(cat_ref[...], pw0c[...], preferred_element_type=jnp.float32)
         + pb0[...])
    h = jnp.maximum(h, 0.0)
    h = jnp.maximum(
        jnp.dot(h, pw1[...], preferred_element_type=jnp.float32) + pb1[...],
        0.0)
    po_ref[...] = (jnp.dot(h, pw2[...], preferred_element_type=jnp.float32)
                   + pb2[...])


def _full(shape):
    return pl.BlockSpec(shape, lambda i: (0,) * len(shape))


def _tc_towers(means, cat_rows,
               q_p0, q_p1, q_p2, q_p3, q_p4, q_p5,
               p_p0t, p_p0c, p_p1, p_p2, p_p3, p_p4, p_p5):
    nblk = _B // _BLK
    return pl.pallas_call(
        _tc_body,
        grid=(nblk,),
        in_specs=[
            pl.BlockSpec((_BLK, _DT), lambda i: (i, 0)),
            pl.BlockSpec((_BLK, _DT), lambda i, n=nblk: (i + n, 0)),
            pl.BlockSpec((_BLK, _DC), lambda i: (i, 0)),
            _full(q_p0.shape), _full(q_p1.shape),
            _full(q_p2.shape), _full(q_p3.shape),
            _full(q_p4.shape), _full(q_p5.shape),
            _full(p_p0t.shape), _full(p_p0c.shape), _full(p_p1.shape),
            _full(p_p2.shape), _full(p_p3.shape),
            _full(p_p4.shape), _full(p_p5.shape),
        ],
        out_specs=[
            pl.BlockSpec((_BLK, _DT), lambda i: (i, 0)),
            pl.BlockSpec((_BLK, _DT), lambda i: (i, 0)),
        ],
        out_shape=[
            jax.ShapeDtypeStruct((_B, _DT), jnp.float32),
            jax.ShapeDtypeStruct((_B, _DT), jnp.float32),
        ],
    )(means, means, cat_rows,
      q_p0, q_p1, q_p2, q_p3, q_p4, q_p5,
      p_p0t, p_p0c, p_p1, p_p2, p_p3, p_p4, p_p5)


def _remap(x, shift):
    rmask = 4096 // (1 << shift) - 1
    return ((x & -4096) | ((x & rmask) << shift)
            | ((x & 4095) >> (12 - shift)))


def kernel(query_text, product_text, category, text_table, cat_table,
           q_p0, q_p1, q_p2, q_p3, q_p4, q_p5,
           p_p0, p_p1, p_p2, p_p3, p_p4, p_p5):
    tflat = _detile(text_table.T)
    cflat = _detile(cat_table.T)

    qr = _remap(query_text, 2)
    pr = _remap(product_text, 2)
    cr = _remap(category, 3)
    q1 = jnp.pad(qr, ((0, 0), (0, _LP - _L))).reshape(-1)
    p1 = jnp.pad(pr, ((0, 0), (0, _LP - _L))).reshape(-1)

    means, cat_rows = _sc_gather()(
        q1, p1, cr,
        tflat.reshape(-1, _DT), cflat.reshape(-1, _DC))

    q_out, p_out = _tc_towers(
        means, cat_rows,
        q_p0, q_p1.reshape(1, -1), q_p2, q_p3.reshape(1, -1),
        q_p4, q_p5.reshape(1, -1),
        p_p0[:_DT], p_p0[_DT:], p_p1.reshape(1, -1),
        p_p2, p_p3.reshape(1, -1), p_p4, p_p5.reshape(1, -1))
    return (q_out, p_out)


# speedup vs baseline: 1.0170x; 1.0170x over previous
import jax
import jax.numpy as jnp
from jax import lax
from jax.experimental import pallas as pl
from jax.experimental.pallas import tpu as pltpu
from jax.experimental.pallas import tpu_sc as plsc

_B = 16384
_L = 50
_DT = 32
_DC = 16
_VT = 1000000
_VC = 100000
_NC = 2
_NS = 16
_NW = _NC * _NS
_LP = 56
_CH = 2
_CL = _CH * _LP
_RPT = _B // _NW
_IPT = _RPT * _LP
_CPT = _RPT // _CH
_CPW = 2 * _CPT
_SPW = 2 * _RPT
_CR2 = _B * _LP // _CL
_NBUF = 4

_CAT_CH = 128
_CAT_PW = _B // _NW
_CAT_CPW = _CAT_PW // _CAT_CH


def _tree_sum(parts):
    while len(parts) > 1:
        nxt = [parts[i] + parts[i + 1] for i in range(0, len(parts) - 1, 2)]
        if len(parts) % 2:
            nxt.append(parts[-1])
        parts = nxt
    return parts[0]


def _detile_body(in_ref, out_ref):
    x = in_ref[...]
    d = x.shape[0]
    ns = 128 // d
    sr = x.shape[1] // ns
    li = lax.broadcasted_iota(jnp.int32, (d, 128), 0)
    lj = lax.broadcasted_iota(jnp.int32, (d, 128), 1)
    acc = jnp.zeros(out_ref.shape, jnp.float32)
    for a in range(ns):
        e = (lj - a * d == li).astype(jnp.float32)
        xa = x[:, a * sr:(a + 1) * sr]
        acc = acc + lax.dot_general(xa, e, (((0,), (0,)), ((), ())),
                                    preferred_element_type=jnp.float32)
    out_ref[...] = acc


def _detile(table_t):
    d, v = table_t.shape
    blkv = 4096
    nblk = pl.cdiv(v, blkv)
    brows = blkv * d // 128
    return pl.pallas_call(
        _detile_body,
        grid=(nblk,),
        in_specs=[pl.BlockSpec((d, blkv), lambda i: (0, i))],
        out_specs=pl.BlockSpec((brows, 128), lambda i: (i, 0)),
        out_shape=jax.ShapeDtypeStruct((nblk * brows, 128), jnp.float32),
    )(table_t)


def _sc_body(qidx_hbm, pidx_hbm, cat_hbm, table_hbm, cat_table_hbm,
             means_hbm, cat_out_hbm,
             idx_v, rows_v, means_v, cidx_v, crows_v, gsems, csem):
    wid = lax.axis_index("s") * _NC + lax.axis_index("c")

    pltpu.sync_copy(qidx_hbm.at[pl.ds(wid * _CPT, _CPT)],
                    idx_v.at[pl.ds(0, _CPT)])
    pltpu.sync_copy(pidx_hbm.at[pl.ds(wid * _CPT, _CPT)],
                    idx_v.at[pl.ds(_CPT, _CPT)])
    pltpu.sync_copy(cat_hbm.at[pl.ds(wid * _CAT_PW, _CAT_PW)], cidx_v)

    for j in range(_CAT_CPW):
        pltpu.async_copy(
            cat_table_hbm.at[cidx_v.at[pl.ds(j * _CAT_CH, _CAT_CH)]],
            crows_v.at[pl.ds(j * _CAT_CH, _CAT_CH)], csem)

    lane = lax.iota(jnp.int32, 16)
    tail_mask = lane >= 14

    def start(c, b):
        pltpu.async_copy(table_hbm.at[idx_v.at[c]], rows_v.at[b], gsems[b])

    def wait(c, b):
        pltpu.make_async_copy(table_hbm.at[idx_v.at[c]], rows_v.at[b],
                              gsems[b]).wait()

    def accum(c, b):
        rb = rows_v.at[b]
        for s in range(_CH):
            base = s * _LP
            lo = _tree_sum([rb[base + j, pl.ds(0, 16)] for j in range(_L)])
            hi = _tree_sum([rb[base + j, pl.ds(16, 16)] for j in range(_L)])
            pc = plsc.all_reduce_population_count(
                idx_v[c, pl.ds(base, 16)] != 0)
            pc = pc + plsc.all_reduce_population_count(
                idx_v[c, pl.ds(base + 16, 16)] != 0)
            pc = pc + plsc.all_reduce_population_count(
                idx_v[c, pl.ds(base + 32, 16)] != 0)
            pc = pc + plsc.all_reduce_population_count(
                (idx_v[c, pl.ds(base + 34, 16)] != 0) & tail_mask)
            inv = 1.0 / jnp.maximum(pc, 1).astype(jnp.float32)
            row = c * _CH + s
            means_v[row, pl.ds(0, 16)] = lo * inv
            means_v[row, pl.ds(16, 16)] = hi * inv

    for b in range(_NBUF - 1):
        start(b, b)

    @pl.loop(0, _CPW, step=_NBUF)
    def _(c0):
        for b in range(_NBUF):
            c = c0 + b
            nxt = c + (_NBUF - 1)

            @pl.when(nxt < _CPW)
            def _():
                start(nxt, (b + _NBUF - 1) % _NBUF)

            wait(c, b)
            accum(c, b)

    half = _SPW // 2
    pltpu.sync_copy(means_v.at[pl.ds(0, half)],
                    means_hbm.at[pl.ds(wid * half, half)])
    pltpu.sync_copy(means_v.at[pl.ds(half, half)],
                    means_hbm.at[pl.ds(_B + wid * half, half)])

    for j in range(_CAT_CPW):
        pltpu.make_async_copy(
            cat_table_hbm.at[cidx_v.at[pl.ds(j * _CAT_CH, _CAT_CH)]],
            crows_v.at[pl.ds(j * _CAT_CH, _CAT_CH)], csem).wait()
    pltpu.sync_copy(crows_v, cat_out_hbm.at[pl.ds(wid * _CAT_PW, _CAT_PW)])


def _sc_gather():
    return pl.kernel(
        _sc_body,
        out_type=(jax.ShapeDtypeStruct((2 * _B, _DT), jnp.float32),
                  jax.ShapeDtypeStruct((_B, _DC), jnp.float32)),
        mesh=plsc.VectorSubcoreMesh(core_axis_name="c", subcore_axis_name="s"),
        compiler_params=pltpu.CompilerParams(use_tc_tiling_on_sc=False,
                                             needs_layout_passes=False),
        scratch_types=[
            pltpu.VMEM((_CPW, _CL), jnp.int32),
            pltpu.VMEM((_NBUF, _CL, _DT), jnp.float32),
            pltpu.VMEM((_SPW, _DT), jnp.float32),
            pltpu.VMEM((_CAT_PW,), jnp.int32),
            pltpu.VMEM((_CAT_PW, _DC), jnp.float32),
            [pltpu.SemaphoreType.DMA] * _NBUF,
            pltpu.SemaphoreType.DMA,
        ],
    )


_BLK = 2048


def _tc_body(qm_ref, pm_ref, cat_ref,
             qw0, qb0, qw1, qb1, qw2, qb2,
             pw0t, pw0c, pb0, pw1, pb1, pw2, pb2,
             qo_ref, po_ref):
    q = qm_ref[...]
    h = jnp.maximum(
        jnp.dot(q, qw0[...], preferred_element_type=jnp.float32) + qb0[...],
        0.0)
    h = jnp.maximum(
        jnp.dot(h, qw1[...], preferred_element_type=jnp.float32) + qb1[...],
        0.0)
    qo_ref[...] = (jnp.dot(h, qw2[...], preferred_element_type=jnp.float32)
                   + qb2[...])

    t = pm_ref[...]
    h = (jnp.dot(t, pw0t[...], preferred_element_type=jnp.float32)
         + jnp.dot(cat_ref[...], pw0c[...], preferred_element_type=jnp.float32)
         + pb0[...])
    h = jnp.maximum(h, 0.0)
    h = jnp.maximum(
        jnp.dot(h, pw1[...], preferred_element_type=jnp.float32) + pb1[...],
        0.0)
    po_ref[...] = (jnp.dot(h, pw2[...], preferred_element_type=jnp.float32)
                   + pb2[...])


def _full(shape):
    return pl.BlockSpec(shape, lambda i: (0,) * len(shape))


def _tc_towers(means, cat_rows,
               q_p0, q_p1, q_p2, q_p3, q_p4, q_p5,
               p_p0t, p_p0c, p_p1, p_p2, p_p3, p_p4, p_p5):
    nblk = _B // _BLK
    return pl.pallas_call(
        _tc_body,
        grid=(nblk,),
        in_specs=[
            pl.BlockSpec((_BLK, _DT), lambda i: (i, 0)),
            pl.BlockSpec((_BLK, _DT), lambda i, n=nblk: (i + n, 0)),
            pl.BlockSpec((_BLK, _DC), lambda i: (i, 0)),
            _full(q_p0.shape), _full(q_p1.shape),
            _full(q_p2.shape), _full(q_p3.shape),
            _full(q_p4.shape), _full(q_p5.shape),
            _full(p_p0t.shape), _full(p_p0c.shape), _full(p_p1.shape),
            _full(p_p2.shape), _full(p_p3.shape),
            _full(p_p4.shape), _full(p_p5.shape),
        ],
        out_specs=[
            pl.BlockSpec((_BLK, _DT), lambda i: (i, 0)),
            pl.BlockSpec((_BLK, _DT), lambda i: (i, 0)),
        ],
        out_shape=[
            jax.ShapeDtypeStruct((_B, _DT), jnp.float32),
            jax.ShapeDtypeStruct((_B, _DT), jnp.float32),
        ],
    )(means, means, cat_rows,
      q_p0, q_p1, q_p2, q_p3, q_p4, q_p5,
      p_p0t, p_p0c, p_p1, p_p2, p_p3, p_p4, p_p5)


def _remap(x, shift):
    rmask = 4096 // (1 << shift) - 1
    return ((x & -4096) | ((x & rmask) << shift)
            | ((x & 4095) >> (12 - shift)))


def kernel(query_text, product_text, category, text_table, cat_table,
           q_p0, q_p1, q_p2, q_p3, q_p4, q_p5,
           p_p0, p_p1, p_p2, p_p3, p_p4, p_p5):
    tflat = _detile(text_table.T)
    cflat = _detile(cat_table.T)

    qr = _remap(query_text, 2)
    pr = _remap(product_text, 2)
    cr = _remap(category, 3)
    q1 = jnp.pad(qr, ((0, 0), (0, _LP - _L))).reshape(_CR2, _CL)
    p1 = jnp.pad(pr, ((0, 0), (0, _LP - _L))).reshape(_CR2, _CL)

    means, cat_rows = _sc_gather()(
        q1, p1, cr,
        tflat.reshape(-1, _DT), cflat.reshape(-1, _DC))

    q_out, p_out = _tc_towers(
        means, cat_rows,
        q_p0, q_p1.reshape(1, -1), q_p2, q_p3.reshape(1, -1),
        q_p4, q_p5.reshape(1, -1),
        p_p0[:_DT], p_p0[_DT:], p_p1.reshape(1, -1),
        p_p2, p_p3.reshape(1, -1), p_p4, p_p5.reshape(1, -1))
    return (q_out, p_out)


# speedup vs baseline: 4.6888x; 4.6104x over previous
import jax
import jax.numpy as jnp
from jax import lax
from jax.experimental import pallas as pl
from jax.experimental.pallas import tpu as pltpu
from jax.experimental.pallas import tpu_sc as plsc

_B = 16384
_L = 50
_DT = 32
_DC = 16
_VT = 1000000
_VC = 100000
_NC = 2
_NS = 16
_NW = _NC * _NS
_LP = 56
_CH = 2
_CL = _CH * _LP
_RPT = _B // _NW
_IPT = _RPT * _LP
_CPT = _RPT // _CH
_CPW = 2 * _CPT
_SPW = 2 * _RPT
_CR2 = _B * _LP // _CL
_NBUF = 4

_CAT_CH = 128
_CAT_PW = _B // _NW
_CAT_CPW = _CAT_PW // _CAT_CH


def _tree_sum(parts):
    while len(parts) > 1:
        nxt = [parts[i] + parts[i + 1] for i in range(0, len(parts) - 1, 2)]
        if len(parts) % 2:
            nxt.append(parts[-1])
        parts = nxt
    return parts[0]


def _detile_body(in_ref, out_ref):
    x = in_ref[...]
    d = x.shape[0]
    ns = 128 // d
    sr = x.shape[1] // ns
    li = lax.broadcasted_iota(jnp.int32, (d, 128), 0)
    lj = lax.broadcasted_iota(jnp.int32, (d, 128), 1)
    acc = jnp.zeros(out_ref.shape, jnp.float32)
    for a in range(ns):
        e = (lj - a * d == li).astype(jnp.float32)
        xa = x[:, a * sr:(a + 1) * sr]
        acc = acc + lax.dot_general(xa, e, (((0,), (0,)), ((), ())),
                                    preferred_element_type=jnp.float32)
    out_ref[...] = acc


def _detile(table_t):
    d, v = table_t.shape
    blkv = 4096
    nblk = pl.cdiv(v, blkv)
    brows = blkv * d // 128
    return pl.pallas_call(
        _detile_body,
        grid=(nblk,),
        in_specs=[pl.BlockSpec((d, blkv), lambda i: (0, i))],
        out_specs=pl.BlockSpec((brows, 128), lambda i: (i, 0)),
        out_shape=jax.ShapeDtypeStruct((nblk * brows, 128), jnp.float32),
    )(table_t)


def _sc_body(qidx_hbm, pidx_hbm, cat_hbm, table_hbm, cat_table_hbm,
             means_hbm, cat_out_hbm,
             idx_v, rows_v, means_v, cidx_v, crows_v, gsems, csem):
    wid = lax.axis_index("s") * _NC + lax.axis_index("c")

    pltpu.sync_copy(qidx_hbm.at[pl.ds(wid * _CPT, _CPT)],
                    idx_v.at[pl.ds(0, _CPT)])
    pltpu.sync_copy(pidx_hbm.at[pl.ds(wid * _CPT, _CPT)],
                    idx_v.at[pl.ds(_CPT, _CPT)])
    pltpu.sync_copy(cat_hbm.at[pl.ds(wid * _CAT_PW, _CAT_PW)], cidx_v)

    for j in range(_CAT_CPW):
        pltpu.async_copy(
            cat_table_hbm.at[cidx_v.at[pl.ds(j * _CAT_CH, _CAT_CH)]],
            crows_v.at[pl.ds(j * _CAT_CH, _CAT_CH)], csem)

    lane = lax.iota(jnp.int32, 16)
    tail_mask = lane >= 14

    def start(c, b):
        pltpu.async_copy(table_hbm.at[idx_v.at[c]], rows_v.at[b], gsems[b])

    def wait(c, b):
        pltpu.make_async_copy(table_hbm.at[idx_v.at[c]], rows_v.at[b],
                              gsems[b]).wait()

    def accum(c, b):
        rb = rows_v.at[b]
        for s in range(_CH):
            base = s * _LP
            lo = _tree_sum([rb[base + j, pl.ds(0, 16)] for j in range(_L)])
            hi = _tree_sum([rb[base + j, pl.ds(16, 16)] for j in range(_L)])
            pc = plsc.all_reduce_population_count(
                idx_v[c, pl.ds(base, 16)] != 0)
            pc = pc + plsc.all_reduce_population_count(
                idx_v[c, pl.ds(base + 16, 16)] != 0)
            pc = pc + plsc.all_reduce_population_count(
                idx_v[c, pl.ds(base + 32, 16)] != 0)
            pc = pc + plsc.all_reduce_population_count(
                (idx_v[c, pl.ds(base + 34, 16)] != 0) & tail_mask)
            inv = 1.0 / jnp.maximum(pc, 1).astype(jnp.float32)
            row = c * _CH + s
            means_v[row, pl.ds(0, 16)] = lo * inv
            means_v[row, pl.ds(16, 16)] = hi * inv

    for b in range(_NBUF - 1):
        start(b, b)

    @pl.loop(0, _CPW, step=_NBUF)
    def _(c0):
        for b in range(_NBUF):
            c = c0 + b
            nxt = c + (_NBUF - 1)

            @pl.when(nxt < _CPW)
            def _():
                start(nxt, (b + _NBUF - 1) % _NBUF)

            wait(c, b)
            accum(c, b)

    half = _SPW // 2
    pltpu.sync_copy(means_v.at[pl.ds(0, half)],
                    means_hbm.at[pl.ds(wid * half, half)])
    pltpu.sync_copy(means_v.at[pl.ds(half, half)],
                    means_hbm.at[pl.ds(_B + wid * half, half)])

    for j in range(_CAT_CPW):
        pltpu.make_async_copy(
            cat_table_hbm.at[cidx_v.at[pl.ds(j * _CAT_CH, _CAT_CH)]],
            crows_v.at[pl.ds(j * _CAT_CH, _CAT_CH)], csem).wait()
    pltpu.sync_copy(crows_v, cat_out_hbm.at[pl.ds(wid * _CAT_PW, _CAT_PW)])


def _sc_gather():
    return pl.kernel(
        _sc_body,
        out_type=(jax.ShapeDtypeStruct((2 * _B, _DT), jnp.float32),
                  jax.ShapeDtypeStruct((_B, _DC), jnp.float32)),
        mesh=plsc.VectorSubcoreMesh(core_axis_name="c", subcore_axis_name="s"),
        compiler_params=pltpu.CompilerParams(use_tc_tiling_on_sc=False,
                                             needs_layout_passes=False),
        scratch_types=[
            pltpu.VMEM((_CPW, _CL), jnp.int32),
            pltpu.VMEM((_NBUF, _CL, _DT), jnp.float32),
            pltpu.VMEM((_SPW, _DT), jnp.float32),
            pltpu.VMEM((_CAT_PW,), jnp.int32),
            pltpu.VMEM((_CAT_PW, _DC), jnp.float32),
            [pltpu.SemaphoreType.DMA] * _NBUF,
            pltpu.SemaphoreType.DMA,
        ],
    )


_BLK = 2048


def _tc_body(qm_ref, pm_ref, cat_ref,
             qw0, qb0, qw1, qb1, qw2, qb2,
             pw0t, pw0c, pb0, pw1, pb1, pw2, pb2,
             qo_ref, po_ref):
    q = qm_ref[...]
    h = jnp.maximum(
        jnp.dot(q, qw0[...], preferred_element_type=jnp.float32) + qb0[...],
        0.0)
    h = jnp.maximum(
        jnp.dot(h, qw1[...], preferred_element_type=jnp.float32) + qb1[...],
        0.0)
    qo_ref[...] = (jnp.dot(h, qw2[...], preferred_element_type=jnp.float32)
                   + qb2[...])

    t = pm_ref[...]
    h = (jnp.dot(t, pw0t[...], preferred_element_type=jnp.float32)
         + jnp.dot(cat_ref[...], pw0c[...], preferred_element_type=jnp.float32)
         + pb0[...])
    h = jnp.maximum(h, 0.0)
    h = jnp.maximum(
        jnp.dot(h, pw1[...], preferred_element_type=jnp.float32) + pb1[...],
        0.0)
    po_ref[...] = (jnp.dot(h, pw2[...], preferred_element_type=jnp.float32)
                   + pb2[...])


def _full(shape):
    return pl.BlockSpec(shape, lambda i: (0,) * len(shape))


def _tc_towers(means, cat_rows,
               q_p0, q_p1, q_p2, q_p3, q_p4, q_p5,
               p_p0t, p_p0c, p_p1, p_p2, p_p3, p_p4, p_p5):
    nblk = _B // _BLK
    return pl.pallas_call(
        _tc_body,
        grid=(nblk,),
        in_specs=[
            pl.BlockSpec((_BLK, _DT), lambda i: (i, 0)),
            pl.BlockSpec((_BLK, _DT), lambda i, n=nblk: (i + n, 0)),
            pl.BlockSpec((_BLK, _DC), lambda i: (i, 0)),
            _full(q_p0.shape), _full(q_p1.shape),
            _full(q_p2.shape), _full(q_p3.shape),
            _full(q_p4.shape), _full(q_p5.shape),
            _full(p_p0t.shape), _full(p_p0c.shape), _full(p_p1.shape),
            _full(p_p2.shape), _full(p_p3.shape),
            _full(p_p4.shape), _full(p_p5.shape),
        ],
        out_specs=[
            pl.BlockSpec((_BLK, _DT), lambda i: (i, 0)),
            pl.BlockSpec((_BLK, _DT), lambda i: (i, 0)),
        ],
        out_shape=[
            jax.ShapeDtypeStruct((_B, _DT), jnp.float32),
            jax.ShapeDtypeStruct((_B, _DT), jnp.float32),
        ],
    )(means, means, cat_rows,
      q_p0, q_p1, q_p2, q_p3, q_p4, q_p5,
      p_p0t, p_p0c, p_p1, p_p2, p_p3, p_p4, p_p5)


def _remap(x, shift):
    rmask = 4096 // (1 << shift) - 1
    return ((x & -4096) | ((x & rmask) << shift)
            | ((x & 4095) >> (12 - shift)))


def kernel(query_text, product_text, category, text_table, cat_table,
           q_p0, q_p1, q_p2, q_p3, q_p4, q_p5,
           p_p0, p_p1, p_p2, p_p3, p_p4, p_p5):
    tflat = _detile(text_table.T)
    cflat = _detile(cat_table.T)

    rows_i = jnp.arange(_B, dtype=jnp.int32)[:, None]
    cols_i = jnp.arange(_LP - _L, dtype=jnp.int32)[None, :]
    spread = ((rows_i * (_LP - _L) + cols_i) * 104729) % _VT
    qr = _remap(jnp.concatenate([query_text, spread], axis=1), 2)
    pr = _remap(jnp.concatenate([product_text, spread + 37], axis=1), 2)
    cr = _remap(category, 3)
    q1 = qr.reshape(_CR2, _CL)
    p1 = pr.reshape(_CR2, _CL)

    means, cat_rows = _sc_gather()(
        q1, p1, cr,
        tflat.reshape(-1, _DT), cflat.reshape(-1, _DC))

    q_out, p_out = _tc_towers(
        means, cat_rows,
        q_p0, q_p1.reshape(1, -1), q_p2, q_p3.reshape(1, -1),
        q_p4, q_p5.reshape(1, -1),
        p_p0[:_DT], p_p0[_DT:], p_p1.reshape(1, -1),
        p_p2, p_p3.reshape(1, -1), p_p4, p_p5.reshape(1, -1))
    return (q_out, p_out)


# speedup vs baseline: 5.8975x; 1.2578x over previous
import jax
import jax.numpy as jnp
from jax import lax
from jax.experimental import pallas as pl
from jax.experimental.pallas import tpu as pltpu
from jax.experimental.pallas import tpu_sc as plsc

_B = 16384
_L = 50
_DT = 32
_DC = 16
_VT = 1000000
_VC = 100000
_NC = 2
_NS = 16
_NW = _NC * _NS
_LP = 56
_CH = 2
_CL = _CH * _LP
_RPT = _B // _NW
_IPT = _RPT * _LP
_CPT = _RPT // _CH
_CPW = 2 * _CPT
_SPW = 2 * _RPT
_CR2 = _B * _LP // _CL
_NBUF = 4

_CAT_CH = 128
_CAT_PW = _B // _NW
_CAT_CPW = _CAT_PW // _CAT_CH


def _tree_sum(parts):
    while len(parts) > 1:
        nxt = [parts[i] + parts[i + 1] for i in range(0, len(parts) - 1, 2)]
        if len(parts) % 2:
            nxt.append(parts[-1])
        parts = nxt
    return parts[0]


def _detile_body(in_ref, out_ref):
    x = in_ref[...]
    d = x.shape[0]
    ns = 128 // d
    sr = x.shape[1] // ns
    li = lax.broadcasted_iota(jnp.int32, (d, 128), 0)
    lj = lax.broadcasted_iota(jnp.int32, (d, 128), 1)
    acc = jnp.zeros(out_ref.shape, jnp.float32)
    for a in range(ns):
        e = (lj - a * d == li).astype(jnp.float32)
        xa = x[:, a * sr:(a + 1) * sr]
        acc = acc + lax.dot_general(xa, e, (((0,), (0,)), ((), ())),
                                    preferred_element_type=jnp.float32)
    out_ref[...] = acc


def _detile(table_t):
    d, v = table_t.shape
    blkv = 16384
    nblk = pl.cdiv(v, blkv)
    brows = blkv * d // 128
    return pl.pallas_call(
        _detile_body,
        grid=(nblk,),
        in_specs=[pl.BlockSpec((d, blkv), lambda i: (0, i))],
        out_specs=pl.BlockSpec((brows, 128), lambda i: (i, 0)),
        out_shape=jax.ShapeDtypeStruct((nblk * brows, 128), jnp.float32),
    )(table_t)


def _sc_body(qidx_hbm, pidx_hbm, cat_hbm, table_hbm, cat_table_hbm,
             means_hbm, cat_out_hbm,
             idx_v, rows_v, means_v, cidx_v, crows_v, gsems, csem):
    wid = lax.axis_index("s") * _NC + lax.axis_index("c")

    pltpu.sync_copy(qidx_hbm.at[pl.ds(wid * _CPT, _CPT)],
                    idx_v.at[pl.ds(0, _CPT)])
    pltpu.sync_copy(pidx_hbm.at[pl.ds(wid * _CPT, _CPT)],
                    idx_v.at[pl.ds(_CPT, _CPT)])
    pltpu.sync_copy(cat_hbm.at[pl.ds(wid * _CAT_PW, _CAT_PW)], cidx_v)

    for j in range(_CAT_CPW):
        pltpu.async_copy(
            cat_table_hbm.at[cidx_v.at[pl.ds(j * _CAT_CH, _CAT_CH)]],
            crows_v.at[pl.ds(j * _CAT_CH, _CAT_CH)], csem)

    lane = lax.iota(jnp.int32, 16)
    tail_mask = lane >= 14

    def start(c, b):
        pltpu.async_copy(table_hbm.at[idx_v.at[c]], rows_v.at[b], gsems[b])

    def wait(c, b):
        pltpu.make_async_copy(table_hbm.at[idx_v.at[c]], rows_v.at[b],
                              gsems[b]).wait()

    def accum(c, b):
        rb = rows_v.at[b]
        for s in range(_CH):
            base = s * _LP
            lo = _tree_sum([rb[base + j, pl.ds(0, 16)] for j in range(_L)])
            hi = _tree_sum([rb[base + j, pl.ds(16, 16)] for j in range(_L)])
            pc = plsc.all_reduce_population_count(
                idx_v[c, pl.ds(base, 16)] != 0)
            pc = pc + plsc.all_reduce_population_count(
                idx_v[c, pl.ds(base + 16, 16)] != 0)
            pc = pc + plsc.all_reduce_population_count(
                idx_v[c, pl.ds(base + 32, 16)] != 0)
            pc = pc + plsc.all_reduce_population_count(
                (idx_v[c, pl.ds(base + 34, 16)] != 0) & tail_mask)
            inv = 1.0 / jnp.maximum(pc, 1).astype(jnp.float32)
            row = c * _CH + s
            means_v[row, pl.ds(0, 16)] = lo * inv
            means_v[row, pl.ds(16, 16)] = hi * inv

    for b in range(_NBUF - 1):
        start(b, b)

    @pl.loop(0, _CPW, step=_NBUF)
    def _(c0):
        for b in range(_NBUF):
            c = c0 + b
            nxt = c + (_NBUF - 1)

            @pl.when(nxt < _CPW)
            def _():
                start(nxt, (b + _NBUF - 1) % _NBUF)

            wait(c, b)
            accum(c, b)

    half = _SPW // 2
    pltpu.sync_copy(means_v.at[pl.ds(0, half)],
                    means_hbm.at[pl.ds(wid * half, half)])
    pltpu.sync_copy(means_v.at[pl.ds(half, half)],
                    means_hbm.at[pl.ds(_B + wid * half, half)])

    for j in range(_CAT_CPW):
        pltpu.make_async_copy(
            cat_table_hbm.at[cidx_v.at[pl.ds(j * _CAT_CH, _CAT_CH)]],
            crows_v.at[pl.ds(j * _CAT_CH, _CAT_CH)], csem).wait()
    pltpu.sync_copy(crows_v, cat_out_hbm.at[pl.ds(wid * _CAT_PW, _CAT_PW)])


def _sc_gather():
    return pl.kernel(
        _sc_body,
        out_type=(jax.ShapeDtypeStruct((2 * _B, _DT), jnp.float32),
                  jax.ShapeDtypeStruct((_B, _DC), jnp.float32)),
        mesh=plsc.VectorSubcoreMesh(core_axis_name="c", subcore_axis_name="s"),
        compiler_params=pltpu.CompilerParams(use_tc_tiling_on_sc=False,
                                             needs_layout_passes=False),
        scratch_types=[
            pltpu.VMEM((_CPW, _CL), jnp.int32),
            pltpu.VMEM((_NBUF, _CL, _DT), jnp.float32),
            pltpu.VMEM((_SPW, _DT), jnp.float32),
            pltpu.VMEM((_CAT_PW,), jnp.int32),
            pltpu.VMEM((_CAT_PW, _DC), jnp.float32),
            [pltpu.SemaphoreType.DMA] * _NBUF,
            pltpu.SemaphoreType.DMA,
        ],
    )


_BLK = 2048


def _tc_body(qm_ref, pm_ref, cat_ref,
             qw0, qb0, qw1, qb1, qw2, qb2,
             pw0t, pw0c, pb0, pw1, pb1, pw2, pb2,
             qo_ref, po_ref):
    q = qm_ref[...]
    h = jnp.maximum(
        jnp.dot(q, qw0[...], preferred_element_type=jnp.float32) + qb0[...],
        0.0)
    h = jnp.maximum(
        jnp.dot(h, qw1[...], preferred_element_type=jnp.float32) + qb1[...],
        0.0)
    qo_ref[...] = (jnp.dot(h, qw2[...], preferred_element_type=jnp.float32)
                   + qb2[...])

    t = pm_ref[...]
    h = (jnp.dot(t, pw0t[...], preferred_element_type=jnp.float32)
         + jnp.dot(cat_ref[...], pw0c[...], preferred_element_type=jnp.float32)
         + pb0[...])
    h = jnp.maximum(h, 0.0)
    h = jnp.maximum(
        jnp.dot(h, pw1[...], preferred_element_type=jnp.float32) + pb1[...],
        0.0)
    po_ref[...] = (jnp.dot(h, pw2[...], preferred_element_type=jnp.float32)
                   + pb2[...])


def _full(shape):
    return pl.BlockSpec(shape, lambda i: (0,) * len(shape))


def _tc_towers(means, cat_rows,
               q_p0, q_p1, q_p2, q_p3, q_p4, q_p5,
               p_p0t, p_p0c, p_p1, p_p2, p_p3, p_p4, p_p5):
    nblk = _B // _BLK
    return pl.pallas_call(
        _tc_body,
        grid=(nblk,),
        in_specs=[
            pl.BlockSpec((_BLK, _DT), lambda i: (i, 0)),
            pl.BlockSpec((_BLK, _DT), lambda i, n=nblk: (i + n, 0)),
            pl.BlockSpec((_BLK, _DC), lambda i: (i, 0)),
            _full(q_p0.shape), _full(q_p1.shape),
            _full(q_p2.shape), _full(q_p3.shape),
            _full(q_p4.shape), _full(q_p5.shape),
            _full(p_p0t.shape), _full(p_p0c.shape), _full(p_p1.shape),
            _full(p_p2.shape), _full(p_p3.shape),
            _full(p_p4.shape), _full(p_p5.shape),
        ],
        out_specs=[
            pl.BlockSpec((_BLK, _DT), lambda i: (i, 0)),
            pl.BlockSpec((_BLK, _DT), lambda i: (i, 0)),
        ],
        out_shape=[
            jax.ShapeDtypeStruct((_B, _DT), jnp.float32),
            jax.ShapeDtypeStruct((_B, _DT), jnp.float32),
        ],
    )(means, means, cat_rows,
      q_p0, q_p1, q_p2, q_p3, q_p4, q_p5,
      p_p0t, p_p0c, p_p1, p_p2, p_p3, p_p4, p_p5)


def _remap(x, shift):
    rmask = 4096 // (1 << shift) - 1
    return ((x & -4096) | ((x & rmask) << shift)
            | ((x & 4095) >> (12 - shift)))


def kernel(query_text, product_text, category, text_table, cat_table,
           q_p0, q_p1, q_p2, q_p3, q_p4, q_p5,
           p_p0, p_p1, p_p2, p_p3, p_p4, p_p5):
    tflat = _detile(text_table.T)
    cflat = _detile(cat_table.T)

    rows_i = jnp.arange(_B, dtype=jnp.int32)[:, None]
    cols_i = jnp.arange(_LP - _L, dtype=jnp.int32)[None, :]
    spread = ((rows_i * (_LP - _L) + cols_i) * 104729) % _VT
    qr = _remap(jnp.concatenate([query_text, spread], axis=1), 2)
    pr = _remap(jnp.concatenate([product_text, spread + 37], axis=1), 2)
    cr = _remap(category, 3)
    q1 = qr.reshape(_CR2, _CL)
    p1 = pr.reshape(_CR2, _CL)

    means, cat_rows = _sc_gather()(
        q1, p1, cr,
        tflat.reshape(-1, _DT), cflat.reshape(-1, _DC))

    q_out, p_out = _tc_towers(
        means, cat_rows,
        q_p0, q_p1.reshape(1, -1), q_p2, q_p3.reshape(1, -1),
        q_p4, q_p5.reshape(1, -1),
        p_p0[:_DT], p_p0[_DT:], p_p1.reshape(1, -1),
        p_p2, p_p3.reshape(1, -1), p_p4, p_p5.reshape(1, -1))
    return (q_out, p_out)


# speedup vs baseline: 5.9057x; 1.0014x over previous
import jax
import jax.numpy as jnp
from jax import lax
from jax.experimental import pallas as pl
from jax.experimental.pallas import tpu as pltpu
from jax.experimental.pallas import tpu_sc as plsc

_B = 16384
_L = 50
_DT = 32
_DC = 16
_VT = 1000000
_VC = 100000
_NC = 2
_NS = 16
_NW = _NC * _NS
_LP = 56
_CH = 2
_CL = _CH * _LP
_RPT = _B // _NW
_IPT = _RPT * _LP
_CPT = _RPT // _CH
_CPW = 2 * _CPT
_SPW = 2 * _RPT
_CR2 = _B * _LP // _CL
_NBUF = 4

_CAT_CH = 128
_CAT_PW = _B // _NW
_CAT_CPW = _CAT_PW // _CAT_CH


def _tree_sum(parts):
    while len(parts) > 1:
        nxt = [parts[i] + parts[i + 1] for i in range(0, len(parts) - 1, 2)]
        if len(parts) % 2:
            nxt.append(parts[-1])
        parts = nxt
    return parts[0]


def _detile_body(in_ref, out_ref):
    x = in_ref[...]
    d = x.shape[0]
    ns = 128 // d
    sr = x.shape[1] // ns
    li = lax.broadcasted_iota(jnp.int32, (d, 128), 0)
    lj = lax.broadcasted_iota(jnp.int32, (d, 128), 1)
    acc = jnp.zeros(out_ref.shape, jnp.float32)
    for a in range(ns):
        e = (lj - a * d == li).astype(jnp.float32)
        xa = x[:, a * sr:(a + 1) * sr]
        acc = acc + lax.dot_general(xa, e, (((0,), (0,)), ((), ())),
                                    preferred_element_type=jnp.float32)
    out_ref[...] = acc


def _detile(table_t):
    d, v = table_t.shape
    blkv = _BLKV
    nblk = pl.cdiv(v, blkv)
    brows = blkv * d // 128
    return pl.pallas_call(
        _detile_body,
        grid=(nblk,),
        in_specs=[pl.BlockSpec((d, blkv), lambda i: (0, i))],
        out_specs=pl.BlockSpec((brows, 128), lambda i: (i, 0)),
        out_shape=jax.ShapeDtypeStruct((nblk * brows, 128), jnp.float32),
    )(table_t)


def _sc_body(qidx_hbm, pidx_hbm, cat_hbm, table_hbm, cat_table_hbm,
             means_hbm, cat_out_hbm,
             idx_v, rows_v, means_v, cidx_v, crows_v, gsems, csem):
    wid = lax.axis_index("s") * _NC + lax.axis_index("c")

    pltpu.sync_copy(qidx_hbm.at[pl.ds(wid * _CPT, _CPT)],
                    idx_v.at[pl.ds(0, _CPT)])
    pltpu.sync_copy(pidx_hbm.at[pl.ds(wid * _CPT, _CPT)],
                    idx_v.at[pl.ds(_CPT, _CPT)])
    pltpu.sync_copy(cat_hbm.at[pl.ds(wid * _CAT_PW, _CAT_PW)], cidx_v)

    for j in range(_CAT_CPW):
        pltpu.async_copy(
            cat_table_hbm.at[cidx_v.at[pl.ds(j * _CAT_CH, _CAT_CH)]],
            crows_v.at[pl.ds(j * _CAT_CH, _CAT_CH)], csem)

    lane = lax.iota(jnp.int32, 16)
    tail_mask = lane >= 14

    def start(c, b):
        pltpu.async_copy(table_hbm.at[idx_v.at[c]], rows_v.at[b], gsems[b])

    def wait(c, b):
        pltpu.make_async_copy(table_hbm.at[idx_v.at[c]], rows_v.at[b],
                              gsems[b]).wait()

    def accum(c, b):
        rb = rows_v.at[b]
        for s in range(_CH):
            base = s * _LP
            lo = _tree_sum([rb[base + j, pl.ds(0, 16)] for j in range(_L)])
            hi = _tree_sum([rb[base + j, pl.ds(16, 16)] for j in range(_L)])
            pc = plsc.all_reduce_population_count(
                idx_v[c, pl.ds(base, 16)] != 0)
            pc = pc + plsc.all_reduce_population_count(
                idx_v[c, pl.ds(base + 16, 16)] != 0)
            pc = pc + plsc.all_reduce_population_count(
                idx_v[c, pl.ds(base + 32, 16)] != 0)
            pc = pc + plsc.all_reduce_population_count(
                (idx_v[c, pl.ds(base + 34, 16)] != 0) & tail_mask)
            inv = 1.0 / jnp.maximum(pc, 1).astype(jnp.float32)
            row = c * _CH + s
            means_v[row, pl.ds(0, 16)] = lo * inv
            means_v[row, pl.ds(16, 16)] = hi * inv

    for b in range(_NBUF - 1):
        start(b, b)

    @pl.loop(0, _CPW, step=_NBUF)
    def _(c0):
        for b in range(_NBUF):
            c = c0 + b
            nxt = c + (_NBUF - 1)

            @pl.when(nxt < _CPW)
            def _():
                start(nxt, (b + _NBUF - 1) % _NBUF)

            wait(c, b)
            accum(c, b)

    half = _SPW // 2
    pltpu.sync_copy(means_v.at[pl.ds(0, half)],
                    means_hbm.at[pl.ds(wid * half, half)])
    pltpu.sync_copy(means_v.at[pl.ds(half, half)],
                    means_hbm.at[pl.ds(_B + wid * half, half)])

    for j in range(_CAT_CPW):
        pltpu.make_async_copy(
            cat_table_hbm.at[cidx_v.at[pl.ds(j * _CAT_CH, _CAT_CH)]],
            crows_v.at[pl.ds(j * _CAT_CH, _CAT_CH)], csem).wait()
    pltpu.sync_copy(crows_v, cat_out_hbm.at[pl.ds(wid * _CAT_PW, _CAT_PW)])


def _sc_gather():
    return pl.kernel(
        _sc_body,
        out_type=(jax.ShapeDtypeStruct((2 * _B, _DT), jnp.float32),
                  jax.ShapeDtypeStruct((_B, _DC), jnp.float32)),
        mesh=plsc.VectorSubcoreMesh(core_axis_name="c", subcore_axis_name="s"),
        compiler_params=pltpu.CompilerParams(use_tc_tiling_on_sc=False,
                                             needs_layout_passes=False),
        scratch_types=[
            pltpu.VMEM((_CPW, _CL), jnp.int32),
            pltpu.VMEM((_NBUF, _CL, _DT), jnp.float32),
            pltpu.VMEM((_SPW, _DT), jnp.float32),
            pltpu.VMEM((_CAT_PW,), jnp.int32),
            pltpu.VMEM((_CAT_PW, _DC), jnp.float32),
            [pltpu.SemaphoreType.DMA] * _NBUF,
            pltpu.SemaphoreType.DMA,
        ],
    )


_BLK = 2048


def _tc_body(qm_ref, pm_ref, cat_ref,
             qw0, qb0, qw1, qb1, qw2, qb2,
             pw0t, pw0c, pb0, pw1, pb1, pw2, pb2,
             qo_ref, po_ref):
    q = qm_ref[...]
    h = jnp.maximum(
        jnp.dot(q, qw0[...], preferred_element_type=jnp.float32) + qb0[...],
        0.0)
    h = jnp.maximum(
        jnp.dot(h, qw1[...], preferred_element_type=jnp.float32) + qb1[...],
        0.0)
    qo_ref[...] = (jnp.dot(h, qw2[...], preferred_element_type=jnp.float32)
                   + qb2[...])

    t = pm_ref[...]
    h = (jnp.dot(t, pw0t[...], preferred_element_type=jnp.float32)
         + jnp.dot(cat_ref[...], pw0c[...], preferred_element_type=jnp.float32)
         + pb0[...])
    h = jnp.maximum(h, 0.0)
    h = jnp.maximum(
        jnp.dot(h, pw1[...], preferred_element_type=jnp.float32) + pb1[...],
        0.0)
    po_ref[...] = (jnp.dot(h, pw2[...], preferred_element_type=jnp.float32)
                   + pb2[...])


def _full(shape):
    return pl.BlockSpec(shape, lambda i: (0,) * len(shape))


def _tc_towers(means, cat_rows,
               q_p0, q_p1, q_p2, q_p3, q_p4, q_p5,
               p_p0t, p_p0c, p_p1, p_p2, p_p3, p_p4, p_p5):
    nblk = _B // _BLK
    return pl.pallas_call(
        _tc_body,
        grid=(nblk,),
        in_specs=[
            pl.BlockSpec((_BLK, _DT), lambda i: (i, 0)),
            pl.BlockSpec((_BLK, _DT), lambda i, n=nblk: (i + n, 0)),
            pl.BlockSpec((_BLK, _DC), lambda i: (i, 0)),
            _full(q_p0.shape), _full(q_p1.shape),
            _full(q_p2.shape), _full(q_p3.shape),
            _full(q_p4.shape), _full(q_p5.shape),
            _full(p_p0t.shape), _full(p_p0c.shape), _full(p_p1.shape),
            _full(p_p2.shape), _full(p_p3.shape),
            _full(p_p4.shape), _full(p_p5.shape),
        ],
        out_specs=[
            pl.BlockSpec((_BLK, _DT), lambda i: (i, 0)),
            pl.BlockSpec((_BLK, _DT), lambda i: (i, 0)),
        ],
        out_shape=[
            jax.ShapeDtypeStruct((_B, _DT), jnp.float32),
            jax.ShapeDtypeStruct((_B, _DT), jnp.float32),
        ],
    )(means, means, cat_rows,
      q_p0, q_p1, q_p2, q_p3, q_p4, q_p5,
      p_p0t, p_p0c, p_p1, p_p2, p_p3, p_p4, p_p5)


_BLKV = 16384


def _remap(x, shift):
    rmask = _BLKV // (1 << shift) - 1
    return ((x & -_BLKV) | ((x & rmask) << shift)
            | ((x & (_BLKV - 1)) >> (14 - shift)))


def kernel(query_text, product_text, category, text_table, cat_table,
           q_p0, q_p1, q_p2, q_p3, q_p4, q_p5,
           p_p0, p_p1, p_p2, p_p3, p_p4, p_p5):
    tflat = _detile(text_table.T)
    cflat = _detile(cat_table.T)

    rows_i = jnp.arange(_B, dtype=jnp.int32)[:, None]
    cols_i = jnp.arange(_LP - _L, dtype=jnp.int32)[None, :]
    spread = ((rows_i * (_LP - _L) + cols_i) * 104729) % _VT
    qr = _remap(jnp.concatenate([query_text, spread], axis=1), 2)
    pr = _remap(jnp.concatenate([product_text, spread + 37], axis=1), 2)
    cr = _remap(category, 3)
    q1 = qr.reshape(_CR2, _CL)
    p1 = pr.reshape(_CR2, _CL)

    means, cat_rows = _sc_gather()(
        q1, p1, cr,
        tflat.reshape(-1, _DT), cflat.reshape(-1, _DC))

    q_out, p_out = _tc_towers(
        means, cat_rows,
        q_p0, q_p1.reshape(1, -1), q_p2, q_p3.reshape(1, -1),
        q_p4, q_p5.reshape(1, -1),
        p_p0[:_DT], p_p0[_DT:], p_p1.reshape(1, -1),
        p_p2, p_p3.reshape(1, -1), p_p4, p_p5.reshape(1, -1))
    return (q_out, p_out)


# speedup vs baseline: 5.9677x; 1.0105x over previous
import jax
import jax.numpy as jnp
from jax import lax
from jax.experimental import pallas as pl
from jax.experimental.pallas import tpu as pltpu
from jax.experimental.pallas import tpu_sc as plsc

_B = 16384
_L = 50
_DT = 32
_DC = 16
_VT = 1000000
_VC = 100000
_NC = 2
_NS = 16
_NW = _NC * _NS
_LP = 56
_CH = 2
_CL = _CH * _LP
_RPT = _B // _NW
_IPT = _RPT * _LP
_CPT = _RPT // _CH
_CPW = 2 * _CPT
_SPW = 2 * _RPT
_CR2 = _B * _LP // _CL
_NBUF = 4

_CAT_CH = 128
_CAT_PW = _B // _NW
_CAT_CPW = _CAT_PW // _CAT_CH


def _tree_sum(parts):
    while len(parts) > 1:
        nxt = [parts[i] + parts[i + 1] for i in range(0, len(parts) - 1, 2)]
        if len(parts) % 2:
            nxt.append(parts[-1])
        parts = nxt
    return parts[0]


def _detile_body(in_ref, out_ref):
    x = in_ref[...]
    d = x.shape[0]
    ns = 128 // d
    sr = x.shape[1] // ns
    li = lax.broadcasted_iota(jnp.int32, (d, 128), 0)
    lj = lax.broadcasted_iota(jnp.int32, (d, 128), 1)
    acc = jnp.zeros(out_ref.shape, jnp.float32)
    for a in range(ns):
        e = (lj - a * d == li).astype(jnp.float32)
        xa = x[:, a * sr:(a + 1) * sr]
        acc = acc + lax.dot_general(xa, e, (((0,), (0,)), ((), ())),
                                    preferred_element_type=jnp.float32)
    out_ref[...] = acc


def _detile(table_t):
    d, v = table_t.shape
    blkv = _BLKV
    nblk = pl.cdiv(v, blkv)
    brows = blkv * d // 128
    return pl.pallas_call(
        _detile_body,
        grid=(nblk,),
        in_specs=[pl.BlockSpec((d, blkv), lambda i: (0, i))],
        out_specs=pl.BlockSpec((brows, 128), lambda i: (i, 0)),
        out_shape=jax.ShapeDtypeStruct((nblk * brows, 128), jnp.float32),
    )(table_t)


def _sc_body(qidx_hbm, pidx_hbm, cat_hbm, table_hbm, cat_table_hbm,
             means_hbm, cat_out_hbm,
             idx_v, rows_v, means_v, cidx_v, crows_v, gsems, csem):
    wid = lax.axis_index("s") * _NC + lax.axis_index("c")

    pltpu.sync_copy(qidx_hbm.at[pl.ds(wid * _CPT, _CPT)],
                    idx_v.at[pl.ds(0, _CPT)])
    pltpu.sync_copy(pidx_hbm.at[pl.ds(wid * _CPT, _CPT)],
                    idx_v.at[pl.ds(_CPT, _CPT)])
    pltpu.sync_copy(cat_hbm.at[pl.ds(wid * _CAT_PW, _CAT_PW)], cidx_v)

    for j in range(_CAT_CPW):
        pltpu.async_copy(
            cat_table_hbm.at[cidx_v.at[pl.ds(j * _CAT_CH, _CAT_CH)]],
            crows_v.at[pl.ds(j * _CAT_CH, _CAT_CH)], csem)

    lane = lax.iota(jnp.int32, 16)
    tail_mask = lane >= 14

    def start(c, b):
        pltpu.async_copy(table_hbm.at[idx_v.at[c]], rows_v.at[b], gsems[b])

    def wait(c, b):
        pltpu.make_async_copy(table_hbm.at[idx_v.at[c]], rows_v.at[b],
                              gsems[b]).wait()

    def accum(c, b):
        rb = rows_v.at[b]
        for s in range(_CH):
            base = s * _LP
            lo = _tree_sum([rb[base + j, pl.ds(0, 16)] for j in range(_L)])
            hi = _tree_sum([rb[base + j, pl.ds(16, 16)] for j in range(_L)])
            pc = plsc.all_reduce_population_count(
                idx_v[c, pl.ds(base, 16)] != 0)
            pc = pc + plsc.all_reduce_population_count(
                idx_v[c, pl.ds(base + 16, 16)] != 0)
            pc = pc + plsc.all_reduce_population_count(
                idx_v[c, pl.ds(base + 32, 16)] != 0)
            pc = pc + plsc.all_reduce_population_count(
                (idx_v[c, pl.ds(base + 34, 16)] != 0) & tail_mask)
            inv = 1.0 / jnp.maximum(pc, 1).astype(jnp.float32)
            row = c * _CH + s
            means_v[row, pl.ds(0, 16)] = lo * inv
            means_v[row, pl.ds(16, 16)] = hi * inv

    for b in range(_NBUF - 1):
        start(b, b)

    @pl.loop(0, _CPW, step=_NBUF)
    def _(c0):
        for b in range(_NBUF):
            c = c0 + b
            nxt = c + (_NBUF - 1)

            @pl.when(nxt < _CPW)
            def _():
                start(nxt, (b + _NBUF - 1) % _NBUF)

            wait(c, b)
            accum(c, b)

    half = _SPW // 2
    pltpu.sync_copy(means_v.at[pl.ds(0, half)],
                    means_hbm.at[pl.ds(wid * half, half)])
    pltpu.sync_copy(means_v.at[pl.ds(half, half)],
                    means_hbm.at[pl.ds(_B + wid * half, half)])

    for j in range(_CAT_CPW):
        pltpu.make_async_copy(
            cat_table_hbm.at[cidx_v.at[pl.ds(j * _CAT_CH, _CAT_CH)]],
            crows_v.at[pl.ds(j * _CAT_CH, _CAT_CH)], csem).wait()
    pltpu.sync_copy(crows_v, cat_out_hbm.at[pl.ds(wid * _CAT_PW, _CAT_PW)])


def _sc_gather():
    return pl.kernel(
        _sc_body,
        out_type=(jax.ShapeDtypeStruct((2 * _B, _DT), jnp.float32),
                  jax.ShapeDtypeStruct((_B, _DC), jnp.float32)),
        mesh=plsc.VectorSubcoreMesh(core_axis_name="c", subcore_axis_name="s"),
        compiler_params=pltpu.CompilerParams(use_tc_tiling_on_sc=False,
                                             needs_layout_passes=False),
        scratch_types=[
            pltpu.VMEM((_CPW, _CL), jnp.int32),
            pltpu.VMEM((_NBUF, _CL, _DT), jnp.float32),
            pltpu.VMEM((_SPW, _DT), jnp.float32),
            pltpu.VMEM((_CAT_PW,), jnp.int32),
            pltpu.VMEM((_CAT_PW, _DC), jnp.float32),
            [pltpu.SemaphoreType.DMA] * _NBUF,
            pltpu.SemaphoreType.DMA,
        ],
    )


_BLK = 2048


def _tc_body(qm_ref, pm_ref, cat_ref,
             qw0, qb0, qw1, qb1, qw2, qb2,
             pw0t, pw0c, pb0, pw1, pb1, pw2, pb2,
             qo_ref, po_ref):
    q = qm_ref[...]
    h = jnp.maximum(
        jnp.dot(q, qw0[...], preferred_element_type=jnp.float32) + qb0[...],
        0.0)
    h = jnp.maximum(
        jnp.dot(h, qw1[...], preferred_element_type=jnp.float32) + qb1[...],
        0.0)
    qo_ref[...] = (jnp.dot(h, qw2[...], preferred_element_type=jnp.float32)
                   + qb2[...])

    t = pm_ref[...]
    h = (jnp.dot(t, pw0t[...], preferred_element_type=jnp.float32)
         + jnp.dot(cat_ref[...], pw0c[...], preferred_element_type=jnp.float32)
         + pb0[...])
    h = jnp.maximum(h, 0.0)
    h = jnp.maximum(
        jnp.dot(h, pw1[...], preferred_element_type=jnp.float32) + pb1[...],
        0.0)
    po_ref[...] = (jnp.dot(h, pw2[...], preferred_element_type=jnp.float32)
                   + pb2[...])


def _full(shape):
    return pl.BlockSpec(shape, lambda i: (0,) * len(shape))


def _tc_towers(means, cat_rows,
               q_p0, q_p1, q_p2, q_p3, q_p4, q_p5,
               p_p0t, p_p0c, p_p1, p_p2, p_p3, p_p4, p_p5):
    nblk = _B // _BLK
    return pl.pallas_call(
        _tc_body,
        grid=(nblk,),
        in_specs=[
            pl.BlockSpec((_BLK, _DT), lambda i: (i, 0)),
            pl.BlockSpec((_BLK, _DT), lambda i, n=nblk: (i + n, 0)),
            pl.BlockSpec((_BLK, _DC), lambda i: (i, 0)),
            _full(q_p0.shape), _full(q_p1.shape),
            _full(q_p2.shape), _full(q_p3.shape),
            _full(q_p4.shape), _full(q_p5.shape),
            _full(p_p0t.shape), _full(p_p0c.shape), _full(p_p1.shape),
            _full(p_p2.shape), _full(p_p3.shape),
            _full(p_p4.shape), _full(p_p5.shape),
        ],
        out_specs=[
            pl.BlockSpec((_BLK, _DT), lambda i: (i, 0)),
            pl.BlockSpec((_BLK, _DT), lambda i: (i, 0)),
        ],
        out_shape=[
            jax.ShapeDtypeStruct((_B, _DT), jnp.float32),
            jax.ShapeDtypeStruct((_B, _DT), jnp.float32),
        ],
    )(means, means, cat_rows,
      q_p0, q_p1, q_p2, q_p3, q_p4, q_p5,
      p_p0t, p_p0c, p_p1, p_p2, p_p3, p_p4, p_p5)


_BLKV = 32768


def _remap(x, shift):
    rmask = _BLKV // (1 << shift) - 1
    return ((x & -_BLKV) | ((x & rmask) << shift)
            | ((x & (_BLKV - 1)) >> (15 - shift)))


def kernel(query_text, product_text, category, text_table, cat_table,
           q_p0, q_p1, q_p2, q_p3, q_p4, q_p5,
           p_p0, p_p1, p_p2, p_p3, p_p4, p_p5):
    tflat = _detile(text_table.T)
    cflat = _detile(cat_table.T)

    rows_i = jnp.arange(_B, dtype=jnp.int32)[:, None]
    cols_i = jnp.arange(_LP - _L, dtype=jnp.int32)[None, :]
    spread = ((rows_i * (_LP - _L) + cols_i) * 104729) % _VT
    qr = _remap(jnp.concatenate([query_text, spread], axis=1), 2)
    pr = _remap(jnp.concatenate([product_text, spread + 37], axis=1), 2)
    cr = _remap(category, 3)
    q1 = qr.reshape(_CR2, _CL)
    p1 = pr.reshape(_CR2, _CL)

    means, cat_rows = _sc_gather()(
        q1, p1, cr,
        tflat.reshape(-1, _DT), cflat.reshape(-1, _DC))

    q_out, p_out = _tc_towers(
        means, cat_rows,
        q_p0, q_p1.reshape(1, -1), q_p2, q_p3.reshape(1, -1),
        q_p4, q_p5.reshape(1, -1),
        p_p0[:_DT], p_p0[_DT:], p_p1.reshape(1, -1),
        p_p2, p_p3.reshape(1, -1), p_p4, p_p5.reshape(1, -1))
    return (q_out, p_out)
